# baseline (speedup 1.0000x reference)
import functools

import jax
import jax.numpy as jnp
from jax import lax
from jax.experimental import pallas as pl
from jax.experimental.pallas import tpu as pltpu
from jax.experimental.pallas import tpu_sc as plsc

_NUM_MOLECULES = 16384
_NUM_ATOMS = 200

_NC = 2
_NS = 16
_NW = _NC * _NS
_ROWS_PER_W = _NUM_MOLECULES // _NW
_WORDS_PER_W = _ROWS_PER_W * _NUM_ATOMS
_LANES = 16
_FULL_VECS = _NUM_ATOMS // _LANES
_TAIL = _NUM_ATOMS - _FULL_VECS * _LANES


def _shift_kernel(spec_hbm, en_hbm, tab_hbm, out_hbm, spec_v, en_v, out_v,
                  tab_v):
    wid = lax.axis_index("s") * _NC + lax.axis_index("c")
    row0 = wid * _ROWS_PER_W
    word0 = row0 * _NUM_ATOMS

    pltpu.sync_copy(tab_hbm, tab_v)
    pltpu.sync_copy(en_hbm.at[pl.ds(row0, _ROWS_PER_W)], en_v)
    pltpu.sync_copy(spec_hbm.at[pl.ds(word0, _WORDS_PER_W)],
                    spec_v.at[pl.ds(0, _WORDS_PER_W)])
    spec_v[pl.ds(_WORDS_PER_W, _LANES)] = jnp.zeros((_LANES,), jnp.int32)

    lane = lax.iota(jnp.int32, _LANES)
    head = lane < _TAIL

    def group_body(g, _):
        def row_body(j, vec):
            base = (g * _LANES + j) * _NUM_ATOMS
            accs = [jnp.zeros((_LANES,), jnp.float32) for _ in range(4)]
            for v in range(_FULL_VECS):
                sv = spec_v[pl.ds(base + v * _LANES, _LANES)]
                gat = plsc.load_gather(tab_v, [sv])
                accs[v % 4] = accs[v % 4] + gat
            sv = spec_v[pl.ds(base + _FULL_VECS * _LANES, _LANES)]
            gat = plsc.load_gather(tab_v, [sv])
            accs[0] = accs[0] + jnp.where(head, gat, 0.0)
            acc = (accs[0] + accs[1]) + (accs[2] + accs[3])
            sae = jnp.sum(acc)
            return jnp.where(lane == j, sae, vec)

        vec = lax.fori_loop(0, _LANES, row_body,
                            jnp.zeros((_LANES,), jnp.float32))
        sl = pl.ds(g * _LANES, _LANES)
        out_v[sl] = vec + en_v[sl]
        return 0

    lax.fori_loop(0, _ROWS_PER_W // _LANES, group_body, 0)

    pltpu.sync_copy(out_v, out_hbm.at[pl.ds(row0, _ROWS_PER_W)])


@jax.jit
def _shifted(species_flat, energies, table16):
    mesh = plsc.VectorSubcoreMesh(core_axis_name="c", subcore_axis_name="s")
    call = functools.partial(
        pl.kernel,
        out_type=jax.ShapeDtypeStruct((_NUM_MOLECULES,), jnp.float32),
        mesh=mesh,
        scratch_types=[
            pltpu.VMEM((_WORDS_PER_W + _LANES,), jnp.int32),
            pltpu.VMEM((_ROWS_PER_W,), jnp.float32),
            pltpu.VMEM((_ROWS_PER_W,), jnp.float32),
            pltpu.VMEM((_LANES,), jnp.float32),
        ],
        compiler_params=pltpu.CompilerParams(needs_layout_passes=False),
    )(_shift_kernel)
    return call(species_flat, energies, table16)


def kernel(species, energies, self_energies):
    table16 = jnp.concatenate(
        [self_energies, jnp.zeros((16 - self_energies.shape[0],),
                                  jnp.float32)])
    shifted = _shifted(species.reshape(-1), energies, table16)
    return species, shifted


# speedup vs baseline: 1.3592x; 1.3592x over previous
import functools

import jax
import jax.numpy as jnp
from jax import lax
from jax.experimental import pallas as pl
from jax.experimental.pallas import tpu as pltpu
from jax.experimental.pallas import tpu_sc as plsc

_NUM_MOLECULES = 16384
_NUM_ATOMS = 200

_NC = 2
_NS = 16
_NW = _NC * _NS
_ROWS_PER_W = _NUM_MOLECULES // _NW
_LANES = 16
_FULL_VECS = _NUM_ATOMS // _LANES
_TAIL = _NUM_ATOMS - _FULL_VECS * _LANES
_CHUNK_ROWS = 128
_N_CHUNKS = _ROWS_PER_W // _CHUNK_ROWS


def _shift_kernel(spec_hbm, en_hbm, tab_hbm, out_hbm, spec_v, en_v, out_v,
                  tab_v, sem):
    wid = lax.axis_index("s") * _NC + lax.axis_index("c")
    row0 = wid * _ROWS_PER_W

    pltpu.sync_copy(tab_hbm, tab_v)
    pltpu.sync_copy(en_hbm.at[pl.ds(row0, _ROWS_PER_W)], en_v)

    lane = lax.iota(jnp.int32, _LANES)
    tail_mask = lane >= (_LANES - _TAIL)

    def chunk_body(c, _):
        buf = c % 2
        pltpu.sync_copy(spec_hbm.at[pl.ds(row0 + c * _CHUNK_ROWS,
                                          _CHUNK_ROWS)],
                        spec_v.at[buf])

        def group_body(g, _):
            def row_body(j, vec):
                r = g * _LANES + j
                accs = [jnp.zeros((_LANES,), jnp.float32) for _ in range(4)]
                for v in range(_FULL_VECS):
                    sv = spec_v[buf, r, pl.ds(v * _LANES, _LANES)]
                    gat = plsc.load_gather(tab_v, [sv])
                    accs[v % 4] = accs[v % 4] + gat
                sv = spec_v[buf, r, pl.ds(_NUM_ATOMS - _LANES, _LANES)]
                gat = plsc.load_gather(tab_v, [sv])
                accs[0] = accs[0] + jnp.where(tail_mask, gat, 0.0)
                acc = (accs[0] + accs[1]) + (accs[2] + accs[3])
                sae = jnp.sum(acc)
                return jnp.where(lane == j, sae, vec)

            vec = lax.fori_loop(0, _LANES, row_body,
                                jnp.zeros((_LANES,), jnp.float32))
            sl = pl.ds(c * _CHUNK_ROWS + g * _LANES, _LANES)
            out_v[sl] = vec + en_v[sl]
            return 0

        lax.fori_loop(0, _CHUNK_ROWS // _LANES, group_body, 0)
        return 0

    lax.fori_loop(0, _N_CHUNKS, chunk_body, 0)

    pltpu.sync_copy(out_v, out_hbm.at[pl.ds(row0, _ROWS_PER_W)])


@jax.jit
def _shifted(species, energies, table16):
    mesh = plsc.VectorSubcoreMesh(core_axis_name="c", subcore_axis_name="s")
    call = functools.partial(
        pl.kernel,
        out_type=jax.ShapeDtypeStruct((_NUM_MOLECULES,), jnp.float32),
        mesh=mesh,
        scratch_types=[
            pltpu.VMEM((2, _CHUNK_ROWS, _NUM_ATOMS), jnp.int32),
            pltpu.VMEM((_ROWS_PER_W,), jnp.float32),
            pltpu.VMEM((_ROWS_PER_W,), jnp.float32),
            pltpu.VMEM((_LANES,), jnp.float32),
            pltpu.SemaphoreType.DMA,
        ],
        compiler_params=pltpu.CompilerParams(needs_layout_passes=False,
                                             use_tc_tiling_on_sc=True),
    )(_shift_kernel)
    return call(species, energies, table16)


def kernel(species, energies, self_energies):
    table16 = jnp.concatenate(
        [self_energies, jnp.zeros((16 - self_energies.shape[0],),
                                  jnp.float32)])
    shifted = _shifted(species, energies, table16)
    return species, shifted


# speedup vs baseline: 1.7733x; 1.3047x over previous
import functools

import jax
import jax.numpy as jnp
from jax import lax
from jax.experimental import pallas as pl
from jax.experimental.pallas import tpu as pltpu
from jax.experimental.pallas import tpu_sc as plsc

_NUM_MOLECULES = 16384
_NUM_ATOMS = 200

_NC = 2
_NS = 16
_NW = _NC * _NS
_LANES = 16
_MOLS_PER_W = _NUM_MOLECULES // _NW
_TILE_MOLS = 128
_N_TILES = _MOLS_PER_W // _TILE_MOLS
_VECS = _TILE_MOLS // _LANES


def _shift_kernel(spec_hbm, en_hbm, tab_hbm, out_hbm, spec_v, en_v, out_v,
                  tab_v):
    wid = lax.axis_index("s") * _NC + lax.axis_index("c")
    mol0 = wid * _MOLS_PER_W

    pltpu.sync_copy(tab_hbm, tab_v)
    pltpu.sync_copy(en_hbm.at[pl.ds(mol0, _MOLS_PER_W)], en_v)

    def panel_body(t, _):
        pltpu.sync_copy(
            spec_hbm.at[:, pl.ds(mol0 + t * _TILE_MOLS, _TILE_MOLS)],
            spec_v)

        def atom_body(a, accs):
            new = []
            for j in range(_VECS):
                sv = spec_v[a, pl.ds(j * _LANES, _LANES)]
                new.append(accs[j] + plsc.load_gather(tab_v, [sv]))
            return tuple(new)

        accs = lax.fori_loop(
            0, _NUM_ATOMS, atom_body,
            tuple(jnp.zeros((_LANES,), jnp.float32) for _ in range(_VECS)))
        for j in range(_VECS):
            sl = pl.ds(t * _TILE_MOLS + j * _LANES, _LANES)
            out_v[sl] = accs[j] + en_v[sl]
        return 0

    lax.fori_loop(0, _N_TILES, panel_body, 0)

    pltpu.sync_copy(out_v, out_hbm.at[pl.ds(mol0, _MOLS_PER_W)])


@jax.jit
def _shifted(spec_t, energies, table16):
    mesh = plsc.VectorSubcoreMesh(core_axis_name="c", subcore_axis_name="s")
    call = functools.partial(
        pl.kernel,
        out_type=jax.ShapeDtypeStruct((_NUM_MOLECULES,), jnp.float32),
        mesh=mesh,
        scratch_types=[
            pltpu.VMEM((_NUM_ATOMS, _TILE_MOLS), jnp.int32),
            pltpu.VMEM((_MOLS_PER_W,), jnp.float32),
            pltpu.VMEM((_MOLS_PER_W,), jnp.float32),
            pltpu.VMEM((_LANES,), jnp.float32),
        ],
        compiler_params=pltpu.CompilerParams(needs_layout_passes=False,
                                             use_tc_tiling_on_sc=True),
    )(_shift_kernel)
    return call(spec_t, energies, table16)


def kernel(species, energies, self_energies):
    table16 = jnp.concatenate(
        [self_energies, jnp.zeros((16 - self_energies.shape[0],),
                                  jnp.float32)])
    shifted = _shifted(species.T, energies, table16)
    return species, shifted


# speedup vs baseline: 2.3414x; 1.3204x over previous
import functools

import jax
import jax.numpy as jnp
from jax import lax
from jax.experimental import pallas as pl
from jax.experimental.pallas import tpu as pltpu
from jax.experimental.pallas import tpu_sc as plsc

_NUM_MOLECULES = 16384
_NUM_ATOMS = 200

_NC = 2
_NS = 16
_NW = _NC * _NS
_LANES = 16
_MOLS_PER_W = _NUM_MOLECULES // _NW
_TILE_MOLS = 128
_N_TILES = _MOLS_PER_W // _TILE_MOLS
_VECS = _TILE_MOLS // _LANES
_NBUF = 4


def _shift_kernel(spec_hbm, en_hbm, tab_hbm, out_hbm, spec_out_hbm,
                  spec_v, en_v, out_v, tab_v,
                  lsem0, lsem1, lsem2, lsem3, wsem0, wsem1, wsem2, wsem3):
    lsems = (lsem0, lsem1, lsem2, lsem3)
    wsems = (wsem0, wsem1, wsem2, wsem3)
    wid = lax.axis_index("s") * _NC + lax.axis_index("c")
    mol0 = wid * _MOLS_PER_W

    pltpu.sync_copy(tab_hbm, tab_v.at[pl.ds(0, 8)])
    pltpu.sync_copy(en_hbm.at[pl.ds(mol0, _MOLS_PER_W)], en_v)

    def panel_slice(t):
        return pl.ds(mol0 + t * _TILE_MOLS, _TILE_MOLS)

    def start_load(t):
        return pltpu.async_copy(spec_hbm.at[:, panel_slice(t)],
                                spec_v.at[t % _NBUF], lsems[t])

    def start_write(t):
        return pltpu.async_copy(spec_v.at[t % _NBUF],
                                spec_out_hbm.at[:, panel_slice(t)], wsems[t])

    def compute(t):
        buf = t % _NBUF

        def atom_body(a, accs):
            new = []
            for j in range(_VECS):
                sv = spec_v[buf, a, pl.ds(j * _LANES, _LANES)]
                new.append(accs[j] + plsc.load_gather(tab_v, [sv]))
            return tuple(new)

        accs = lax.fori_loop(
            0, _NUM_ATOMS, atom_body,
            tuple(jnp.zeros((_LANES,), jnp.float32) for _ in range(_VECS)))
        for j in range(_VECS):
            sl = pl.ds(t * _TILE_MOLS + j * _LANES, _LANES)
            out_v[sl] = accs[j] + en_v[sl]

    loads = [start_load(t) for t in range(_N_TILES)]
    writes = []
    for t in range(_N_TILES):
        loads[t].wait()
        compute(t)
        writes.append(start_write(t))
    for w in writes:
        w.wait()

    pltpu.sync_copy(out_v, out_hbm.at[pl.ds(mol0, _MOLS_PER_W)])


@jax.jit
def _shifted(spec_t, energies, self_energies):
    mesh = plsc.VectorSubcoreMesh(core_axis_name="c", subcore_axis_name="s")
    call = functools.partial(
        pl.kernel,
        out_type=[
            jax.ShapeDtypeStruct((_NUM_MOLECULES,), jnp.float32),
            jax.ShapeDtypeStruct((_NUM_ATOMS, _NUM_MOLECULES), jnp.int32),
        ],
        mesh=mesh,
        scratch_types=[
            pltpu.VMEM((_NBUF, _NUM_ATOMS, _TILE_MOLS), jnp.int32),
            pltpu.VMEM((_MOLS_PER_W,), jnp.float32),
            pltpu.VMEM((_MOLS_PER_W,), jnp.float32),
            pltpu.VMEM((_LANES,), jnp.float32),
        ] + [pltpu.SemaphoreType.DMA] * 8,
        compiler_params=pltpu.CompilerParams(needs_layout_passes=False,
                                             use_tc_tiling_on_sc=True),
    )(_shift_kernel)
    return call(spec_t, energies, self_energies)


def kernel(species, energies, self_energies):
    shifted, spec_out = _shifted(species.T, energies, self_energies)
    return spec_out.T, shifted


# speedup vs baseline: 2.3730x; 1.0135x over previous
import functools

import jax
import jax.numpy as jnp
from jax import lax
from jax.experimental import pallas as pl
from jax.experimental.pallas import tpu as pltpu
from jax.experimental.pallas import tpu_sc as plsc

_NUM_MOLECULES = 16384
_NUM_ATOMS = 200

_NC = 2
_NS = 16
_NW = _NC * _NS
_LANES = 16
_MOLS_PER_W = _NUM_MOLECULES // _NW
_TILE_MOLS = 128
_N_TILES = _MOLS_PER_W // _TILE_MOLS
_VECS = _TILE_MOLS // _LANES
_QUAD = 4
_NSPEC = 8


def _shift_kernel(spec_hbm, en_hbm, tab_hbm, out_hbm, spec_out_hbm,
                  spec_v, en_v, out_v, tab_v, tab4_v,
                  lsem0, lsem1, lsem2, lsem3, wsem0, wsem1, wsem2, wsem3):
    lsems = (lsem0, lsem1, lsem2, lsem3)
    wsems = (wsem0, wsem1, wsem2, wsem3)
    wid = lax.axis_index("s") * _NC + lax.axis_index("c")
    mol0 = wid * _MOLS_PER_W

    def panel_slice(t):
        return pl.ds(mol0 + t * _TILE_MOLS, _TILE_MOLS)

    def start_load(t):
        return pltpu.async_copy(spec_hbm.at[:, panel_slice(t)],
                                spec_v.at[t], lsems[t])

    def start_write(t):
        return pltpu.async_copy(spec_v.at[t],
                                spec_out_hbm.at[:, panel_slice(t)], wsems[t])

    loads = [start_load(t) for t in range(_N_TILES)]

    pltpu.sync_copy(tab_hbm, tab_v.at[pl.ds(0, _NSPEC)])
    pltpu.sync_copy(en_hbm.at[pl.ds(mol0, _MOLS_PER_W)], en_v)

    lane = lax.iota(jnp.int32, _LANES)
    gd = plsc.load_gather(tab_v, [lane & 7])
    chalf = lane >> 3

    def tab_body(k, _):
        a = lax.shift_right_logical(k, 5)
        b = lax.shift_right_logical(k, 2) & 7
        c0 = (k & 3) * 2
        ga = plsc.load_gather(tab_v, [jnp.broadcast_to(a, (_LANES,))])
        gb = plsc.load_gather(tab_v, [jnp.broadcast_to(b, (_LANES,))])
        gc = plsc.load_gather(tab_v, [jnp.broadcast_to(c0, (_LANES,)) + chalf])
        tab4_v[pl.ds(k * _LANES, _LANES)] = (ga + gb) + (gc + gd)
        return 0

    lax.fori_loop(0, _NSPEC ** _QUAD // _LANES, tab_body, 0)

    def compute(t):
        def quad_body(q, accs):
            a = q * _QUAD
            new = []
            for j in range(_VECS):
                sl = pl.ds(j * _LANES, _LANES)
                s0 = spec_v[t, a, sl]
                s1 = spec_v[t, a + 1, sl]
                s2 = spec_v[t, a + 2, sl]
                s3 = spec_v[t, a + 3, sl]
                idx = ((lax.shift_left(s0, 9) | lax.shift_left(s1, 6))
                       | (lax.shift_left(s2, 3) | s3))
                new.append(accs[j] + plsc.load_gather(tab4_v, [idx]))
            return tuple(new)

        accs = lax.fori_loop(
            0, _NUM_ATOMS // _QUAD, quad_body,
            tuple(jnp.zeros((_LANES,), jnp.float32) for _ in range(_VECS)))
        for j in range(_VECS):
            sl = pl.ds(t * _TILE_MOLS + j * _LANES, _LANES)
            out_v[sl] = accs[j] + en_v[sl]

    writes = []
    for t in range(_N_TILES):
        loads[t].wait()
        compute(t)
        writes.append(start_write(t))
    for w in writes:
        w.wait()

    pltpu.sync_copy(out_v, out_hbm.at[pl.ds(mol0, _MOLS_PER_W)])


@jax.jit
def _shifted(spec_t, energies, self_energies):
    mesh = plsc.VectorSubcoreMesh(core_axis_name="c", subcore_axis_name="s")
    call = functools.partial(
        pl.kernel,
        out_type=[
            jax.ShapeDtypeStruct((_NUM_MOLECULES,), jnp.float32),
            jax.ShapeDtypeStruct((_NUM_ATOMS, _NUM_MOLECULES), jnp.int32),
        ],
        mesh=mesh,
        scratch_types=[
            pltpu.VMEM((_N_TILES, _NUM_ATOMS, _TILE_MOLS), jnp.int32),
            pltpu.VMEM((_MOLS_PER_W,), jnp.float32),
            pltpu.VMEM((_MOLS_PER_W,), jnp.float32),
            pltpu.VMEM((_LANES,), jnp.float32),
            pltpu.VMEM((_NSPEC ** _QUAD,), jnp.float32),
        ] + [pltpu.SemaphoreType.DMA] * 8,
        compiler_params=pltpu.CompilerParams(needs_layout_passes=False,
                                             use_tc_tiling_on_sc=True),
    )(_shift_kernel)
    return call(spec_t, energies, self_energies)


def kernel(species, energies, self_energies):
    shifted, spec_out = _shifted(species.T, energies, self_energies)
    return spec_out.T, shifted
